# baseline (speedup 1.0000x reference)
import functools

import jax
import jax.numpy as jnp
from jax import lax
from jax.experimental import pallas as pl
from jax.experimental.pallas import tpu as pltpu
from jax.experimental.pallas import tpu_sc as plsc

NC = 2
NS = 16
LANES = 16

MAC = 1024
ZR = 128


def _sc_spmm(xT, rows1, cols1, w32, out_features):
    nnz = w32.shape[0]
    n_tiles = NC * NS
    pt = nnz // n_tiles
    n_mac = pt // MAC
    rpt = out_features // NS

    mesh = plsc.VectorSubcoreMesh(core_axis_name="c", subcore_axis_name="s")

    @functools.partial(
        pl.kernel,
        out_type=jax.ShapeDtypeStruct((NC, out_features, LANES), jnp.float32),
        mesh=mesh,
        scratch_types=[
            pltpu.VMEM((MAC,), jnp.int32),
            pltpu.VMEM((MAC,), jnp.int32),
            pltpu.VMEM((MAC,), jnp.int32),
            pltpu.VMEM((MAC,), jnp.int32),
            pltpu.VMEM((MAC,), jnp.int32),
            pltpu.VMEM((MAC,), jnp.int32),
            pltpu.VMEM((MAC,), jnp.int32),
            pltpu.VMEM((MAC,), jnp.int32),
            pltpu.VMEM((4, MAC), jnp.float32),
            pltpu.VMEM((MAC, LANES), jnp.float32),
            pltpu.VMEM((MAC, LANES), jnp.float32),
            pltpu.VMEM((ZR, LANES), jnp.float32),
            pltpu.VMEM_SHARED((out_features, LANES), jnp.float32),
            pltpu.SemaphoreType.DMA,
            pltpu.SemaphoreType.DMA,
            pltpu.SemaphoreType.DMA,
            pltpu.SemaphoreType.DMA,
            pltpu.SemaphoreType.DMA,
            pltpu.SemaphoreType.DMA,
            pltpu.SemaphoreType.DMA,
            pltpu.SemaphoreType.DMA,
        ],
        compiler_params=pltpu.CompilerParams(use_tc_tiling_on_sc=False),
    )
    def spmm(xT_hbm, rows_hbm, cols_hbm, w_hbm, out_hbm,
             r0, r1, r2, r3, c0, c1, c2, c3, w_v, ga, gb, z_v, acc,
             l0, l1, l2, l3, ge0, ge1, se0, se1):
        rows_b = (r0, r1, r2, r3)
        cols_b = (c0, c1, c2, c3)
        g_b = (ga, gb)
        lsem = (l0, l1, l2, l3)
        gsem = (ge0, ge1)
        ssem = (se0, se1)

        core = lax.axis_index("c")
        sub = lax.axis_index("s")
        wid = sub * NC + core

        def zfill(i, c):
            z_v[i] = jnp.zeros((LANES,), jnp.float32)
            return c

        lax.fori_loop(0, ZR, zfill, 0, unroll=8)
        for q in range(rpt // ZR):
            pltpu.sync_copy(z_v, acc.at[pl.ds(sub * rpt + q * ZR, ZR)])
        plsc.subcore_barrier()

        mac0 = wid * n_mac

        def fire_linear(slot, m):
            pltpu.async_copy(cols_hbm.at[pl.ds(m * MAC, MAC)],
                             cols_b[slot], lsem[slot])
            pltpu.async_copy(rows_hbm.at[pl.ds(m * MAC, MAC)],
                             rows_b[slot], lsem[slot])
            pltpu.async_copy(w_hbm.at[pl.ds(m * MAC, MAC)],
                             w_v.at[slot], lsem[slot])

        def wait_linear(slot):
            pltpu.make_async_copy(cols_hbm.at[pl.ds(0, MAC)],
                                  cols_b[slot], lsem[slot]).wait()
            pltpu.make_async_copy(rows_hbm.at[pl.ds(0, MAC)],
                                  rows_b[slot], lsem[slot]).wait()
            pltpu.make_async_copy(w_hbm.at[pl.ds(0, MAC)],
                                  w_v.at[slot], lsem[slot]).wait()

        def fire_gather(slot, par):
            pltpu.async_copy(xT_hbm.at[cols_b[slot]], g_b[par], gsem[par])

        def wait_gather(slot, par):
            pltpu.make_async_copy(xT_hbm.at[cols_b[slot]], g_b[par],
                                  gsem[par]).wait()

        def fire_scatter(slot, par):
            pltpu.async_copy(g_b[par], acc.at[rows_b[slot]], ssem[par],
                             add=True)

        def drain_scatter(slot, par):
            pltpu.make_async_copy(g_b[par], acc.at[rows_b[slot]],
                                  ssem[par]).wait()

        def compute(slot, par):
            g_v = g_b[par]

            def step(t, c):
                base = t * LANES
                w16 = w_v[slot, pl.ds(base, LANES)]
                for l in range(LANES):
                    g_v[base + l] = g_v[base + l] * w16[l]
                return c

            lax.fori_loop(0, MAC // LANES, step, 0, unroll=2)

        fire_linear(0, mac0)
        fire_linear(1, mac0 + 1)

        n_t = n_mac // 4

        def body(t, carry):
            for u in range(4):
                c = 4 * t + u
                su = u
                sg = u % 2
                pu = (u - 1) % 4
                pg = (u - 1) % 2
                qu = (u + 2) % 4

                if u < 2:
                    @pl.when(t > 0)
                    def _():
                        drain_scatter(qu, sg)
                else:
                    drain_scatter(qu, sg)

                if u < 2:
                    fire_linear(qu, mac0 + c + 2)
                else:
                    @pl.when(t < n_t - 1)
                    def _():
                        fire_linear(qu, mac0 + c + 2)

                wait_linear(su)

                fire_gather(su, sg)

                if u == 0:
                    @pl.when(t > 0)
                    def _():
                        wait_gather(pu, pg)
                        compute(pu, pg)
                        fire_scatter(pu, pg)
                else:
                    wait_gather(pu, pg)
                    compute(pu, pg)
                    fire_scatter(pu, pg)
            return carry

        lax.fori_loop(0, n_t, body, 0)

        wait_gather(3, 1)
        compute(3, 1)
        fire_scatter(3, 1)
        drain_scatter(2, 0)
        drain_scatter(3, 1)

        plsc.subcore_barrier()
        pltpu.sync_copy(acc.at[pl.ds(sub * rpt, rpt)],
                        out_hbm.at[core, pl.ds(sub * rpt, rpt)])

    return spmm(xT, rows1, cols1, w32)


def _prep(x, in_features):
    cb = 2048

    def body(x_ref, xt_ref):
        xt_ref[...] = x_ref[...].T

    return pl.pallas_call(
        body,
        grid=(in_features // cb,),
        in_specs=[pl.BlockSpec((LANES, cb), lambda i: (0, i))],
        out_specs=pl.BlockSpec((cb, LANES), lambda i: (i, 0)),
        out_shape=jax.ShapeDtypeStruct((in_features, LANES), jnp.float32),
    )(x)


def _combine(parts, out_features):
    cb = 2048

    def body(p_ref, o_ref):
        o_ref[...] = (p_ref[0] + p_ref[1]).T

    return pl.pallas_call(
        body,
        grid=(out_features // cb,),
        in_specs=[pl.BlockSpec((NC, cb, LANES), lambda i: (0, i, 0))],
        out_specs=pl.BlockSpec((LANES, cb), lambda i: (0, i)),
        out_shape=jax.ShapeDtypeStruct((LANES, out_features), jnp.float32),
    )(parts)


def kernel(x, weights, coords):
    batch, in_features = x.shape
    out_features = in_features
    nnz = weights.shape[0]

    rows1 = coords[:, 0]
    cols1 = coords[:, 1]
    w32 = weights.astype(jnp.float32)
    xT = _prep(x.astype(jnp.float32), in_features)

    parts = _sc_spmm(xT, rows1, cols1, w32, out_features)
    out = _combine(parts, out_features)
    return out.astype(x.dtype)


# speedup vs baseline: 1.1686x; 1.1686x over previous
import functools

import jax
import jax.numpy as jnp
from jax import lax
from jax.experimental import pallas as pl
from jax.experimental.pallas import tpu as pltpu
from jax.experimental.pallas import tpu_sc as plsc

NC = 2
NS = 16
LANES = 16

MAC = 512
NSL = 8
NGB = 4
ZR = 128


def _sc_spmm(xT, rows1, cols1, w32, out_features):
    nnz = w32.shape[0]
    n_tiles = NC * NS
    pt = nnz // n_tiles
    n_mac = pt // MAC
    rpt = out_features // NS
    n_t = n_mac // NSL

    mesh = plsc.VectorSubcoreMesh(core_axis_name="c", subcore_axis_name="s")

    scratch = (
        [pltpu.VMEM((MAC,), jnp.int32) for _ in range(NSL)]
        + [pltpu.VMEM((MAC,), jnp.int32) for _ in range(NSL)]
        + [pltpu.VMEM((NSL, MAC), jnp.float32)]
        + [pltpu.VMEM((MAC, LANES), jnp.float32) for _ in range(NGB)]
        + [pltpu.VMEM((ZR, LANES), jnp.float32)]
        + [pltpu.VMEM_SHARED((out_features, LANES), jnp.float32)]
        + [pltpu.SemaphoreType.DMA for _ in range(NSL + 2 * NGB)]
    )

    @functools.partial(
        pl.kernel,
        out_type=jax.ShapeDtypeStruct((NC, out_features, LANES), jnp.float32),
        mesh=mesh,
        scratch_types=scratch,
        compiler_params=pltpu.CompilerParams(use_tc_tiling_on_sc=False),
    )
    def spmm(xT_hbm, rows_hbm, cols_hbm, w_hbm, out_hbm, *refs):
        rows_b = refs[0:NSL]
        cols_b = refs[NSL:2 * NSL]
        w_v = refs[2 * NSL]
        g_b = refs[2 * NSL + 1:2 * NSL + 1 + NGB]
        z_v = refs[2 * NSL + 1 + NGB]
        acc = refs[2 * NSL + 2 + NGB]
        sems = refs[2 * NSL + 3 + NGB:]
        lsem = sems[0:NSL]
        gsem = sems[NSL:NSL + NGB]
        ssem = sems[NSL + NGB:]

        core = lax.axis_index("c")
        sub = lax.axis_index("s")
        wid = sub * NC + core

        def zfill(i, c):
            z_v[i] = jnp.zeros((LANES,), jnp.float32)
            return c

        lax.fori_loop(0, ZR, zfill, 0, unroll=8)
        for q in range(rpt // ZR):
            pltpu.sync_copy(z_v, acc.at[pl.ds(sub * rpt + q * ZR, ZR)])
        plsc.subcore_barrier()

        mac0 = wid * n_mac

        def fire_linear(slot, m):
            pltpu.async_copy(cols_hbm.at[pl.ds(m * MAC, MAC)],
                             cols_b[slot], lsem[slot])
            pltpu.async_copy(rows_hbm.at[pl.ds(m * MAC, MAC)],
                             rows_b[slot], lsem[slot])
            pltpu.async_copy(w_hbm.at[pl.ds(m * MAC, MAC)],
                             w_v.at[slot], lsem[slot])

        def wait_linear(slot):
            pltpu.make_async_copy(cols_hbm.at[pl.ds(0, MAC)],
                                  cols_b[slot], lsem[slot]).wait()
            pltpu.make_async_copy(rows_hbm.at[pl.ds(0, MAC)],
                                  rows_b[slot], lsem[slot]).wait()
            pltpu.make_async_copy(w_hbm.at[pl.ds(0, MAC)],
                                  w_v.at[slot], lsem[slot]).wait()

        def fire_gather(slot, par):
            pltpu.async_copy(xT_hbm.at[cols_b[slot]], g_b[par], gsem[par])

        def wait_gather(slot, par):
            pltpu.make_async_copy(xT_hbm.at[cols_b[slot]], g_b[par],
                                  gsem[par]).wait()

        def fire_scatter(slot, par):
            pltpu.async_copy(g_b[par], acc.at[rows_b[slot]], ssem[par],
                             add=True)

        def drain_scatter(slot, par):
            pltpu.make_async_copy(g_b[par], acc.at[rows_b[slot]],
                                  ssem[par]).wait()

        def compute(slot, par):
            g_v = g_b[par]

            def step(t, c):
                base = t * LANES
                w16 = w_v[slot, pl.ds(base, LANES)]
                for l in range(LANES):
                    g_v[base + l] = g_v[base + l] * w16[l]
                return c

            lax.fori_loop(0, MAC // LANES, step, 0, unroll=2)

        for m in range(4):
            fire_linear(m, mac0 + m)

        def body(t, carry):
            for u in range(NSL):
                c = NSL * t + u
                su = u
                sg = u % NGB
                pu2 = (u - 2) % NSL
                pg2 = (u - 2) % NGB
                qu = (u + 4) % NSL

                if u < 4:
                    @pl.when(t > 0)
                    def _():
                        drain_scatter(qu, sg)
                else:
                    drain_scatter(qu, sg)

                if u < 4:
                    fire_linear(qu, mac0 + c + 4)
                else:
                    @pl.when(t < n_t - 1)
                    def _():
                        fire_linear(qu, mac0 + c + 4)

                wait_linear(su)

                fire_gather(su, sg)

                if u < 2:
                    @pl.when(t > 0)
                    def _():
                        wait_gather(pu2, pg2)
                        compute(pu2, pg2)
                        fire_scatter(pu2, pg2)
                else:
                    wait_gather(pu2, pg2)
                    compute(pu2, pg2)
                    fire_scatter(pu2, pg2)
            return carry

        lax.fori_loop(0, n_t, body, 0)

        for last in (n_mac - 2, n_mac - 1):
            slot = last % NSL
            par = last % NGB
            wait_gather(slot, par)
            compute(slot, par)
            fire_scatter(slot, par)
        for last in range(n_mac - 4, n_mac):
            drain_scatter(last % NSL, last % NGB)

        plsc.subcore_barrier()
        pltpu.sync_copy(acc.at[pl.ds(sub * rpt, rpt)],
                        out_hbm.at[core, pl.ds(sub * rpt, rpt)])

    return spmm(xT, rows1, cols1, w32)


def _combine(parts, out_features):
    cb = 2048

    def body(p_ref, o_ref):
        o_ref[...] = (p_ref[0] + p_ref[1]).T

    return pl.pallas_call(
        body,
        grid=(out_features // cb,),
        in_specs=[pl.BlockSpec((NC, cb, LANES), lambda i: (0, i, 0))],
        out_specs=pl.BlockSpec((LANES, cb), lambda i: (0, i)),
        out_shape=jax.ShapeDtypeStruct((LANES, out_features), jnp.float32),
    )(parts)


def kernel(x, weights, coords):
    batch, in_features = x.shape
    out_features = in_features
    nnz = weights.shape[0]

    rows1 = coords[:, 0]
    cols1 = coords[:, 1]
    w32 = weights.astype(jnp.float32)
    xT = x.astype(jnp.float32).T

    parts = _sc_spmm(xT, rows1, cols1, w32, out_features)
    out = _combine(parts, out_features)
    return out.astype(x.dtype)
